# Initial kernel scaffold; baseline (speedup 1.0000x reference)
import jax
import jax.numpy as jnp
from jax.experimental import pallas as pl


def kernel(result_features, query_feature, W1, b1, a1, W2, b2, a2, W3, b3, W_ih, W_hh, b_ih, b_hh):
    raise NotImplementedError("write your pallas kernel here")


# speedup vs baseline: 6.7962x; 6.7962x over previous
import jax
import jax.numpy as jnp
from jax.experimental import pallas as pl
from jax.experimental.pallas import tpu as pltpu

FD = 128
RN = 50
STEPS = 10
B = 256
H2 = 512
LANES = 64


def _mega_kernel(rf_ref, qf_ref, w1s_ref, w1f_ref, b1_ref, a1_ref,
                 w2_ref, b2_ref, a2_ref, w3_ref,
                 wih_ref, whh_ref, bih_ref, bhh_ref,
                 ids_ref, pis_ref, p_ref):
    a1 = a1_ref[...]
    a2 = a2_ref[...]
    b1 = b1_ref[...]
    b2 = b2_ref[...]
    w3 = w3_ref[...]
    bih = bih_ref[...]
    bhh = bhh_ref[...]
    w2 = w2_ref[...]

    p_ref[...] = jnp.dot(rf_ref[...].astype(jnp.bfloat16),
                         w1f_ref[...].astype(jnp.bfloat16),
                         preferred_element_type=jnp.float32) + b1

    lane = jax.lax.broadcasted_iota(jnp.int32, (B, LANES), 1)
    neg = jnp.float32(-1e30)

    state = qf_ref[...]
    chosen = jnp.zeros((B, LANES), jnp.bool_)

    w1s = w1s_ref[...].astype(jnp.bfloat16)
    w2b = w2.astype(jnp.bfloat16)
    w3b = w3.astype(jnp.bfloat16)
    wih = wih_ref[...].astype(jnp.bfloat16)
    whh = whh_ref[...].astype(jnp.bfloat16)

    for t in range(STEPS):
        s = jnp.dot(state.astype(jnp.bfloat16), w1s,
                    preferred_element_type=jnp.float32)

        def cand_body(a, acc, s=s):
            pa = p_ref[pl.ds(a * B, B), :]
            h1 = pa + s
            h1 = jnp.where(h1 >= 0, h1, h1 * a1)
            h2 = jnp.dot(h1.astype(jnp.bfloat16), w2b,
                         preferred_element_type=jnp.float32) + b2
            h2 = jnp.where(h2 >= 0, h2, h2 * a2)
            lg = jnp.sum(h2.astype(jnp.bfloat16).astype(jnp.float32)
                         * w3b.astype(jnp.float32),
                         axis=1, keepdims=True)
            return acc + jnp.where(lane == a, lg, 0.0)

        logits = jax.lax.fori_loop(0, RN, cand_body,
                                   jnp.zeros((B, LANES), jnp.float32))

        masked = jnp.where(jnp.logical_or(chosen, lane >= RN), neg, logits)
        mx = jnp.max(masked, axis=1, keepdims=True)
        amax = jnp.min(jnp.where(masked >= mx, lane, LANES),
                       axis=1, keepdims=True)
        denom = jnp.sum(jnp.exp(masked - mx), axis=1, keepdims=True)
        pi = 1.0 / denom

        chosen = jnp.logical_or(chosen, lane == amax)

        def gather_body(a, acc):
            row = rf_ref[pl.ds(a * B, B), :]
            return jnp.where(amax == a, row, acc)

        crf = jax.lax.fori_loop(0, RN, gather_body,
                                jnp.zeros((B, FD), jnp.float32))

        gi = jnp.dot(crf.astype(jnp.bfloat16), wih,
                     preferred_element_type=jnp.float32) + bih
        gh = jnp.dot(state.astype(jnp.bfloat16), whh,
                     preferred_element_type=jnp.float32) + bhh
        r = jax.nn.sigmoid(gi[:, :FD] + gh[:, :FD])
        z = jax.nn.sigmoid(gi[:, FD:2 * FD] + gh[:, FD:2 * FD])
        n = jnp.tanh(gi[:, 2 * FD:] + r * gh[:, 2 * FD:])
        state = (1.0 - z) * n + z * state

        ids_ref[:, t:t + 1] = amax.astype(jnp.float32)
        pis_ref[:, t:t + 1] = pi


def kernel(result_features, query_feature, W1, b1, a1, W2, b2, a2, W3, b3,
           W_ih, W_hh, b_ih, b_hh):
    del b3
    rf = result_features.reshape(RN * B, FD)
    qf = query_feature.reshape(B, FD)
    w1s = W1[:, :FD].T
    w1f = W1[:, FD:].T
    out_shape = (jax.ShapeDtypeStruct((B, STEPS), jnp.float32),
                 jax.ShapeDtypeStruct((B, STEPS), jnp.float32))
    ids, pis = pl.pallas_call(
        _mega_kernel,
        out_shape=out_shape,
        scratch_shapes=[pltpu.VMEM((RN * B, FD), jnp.float32)],
    )(rf, qf, w1s, w1f,
      b1.reshape(1, FD), a1.reshape(1, 1),
      W2.T, b2.reshape(1, H2), a2.reshape(1, 1), W3,
      W_ih.T, W_hh.T, b_ih.reshape(1, 3 * FD), b_hh.reshape(1, 3 * FD))
    return ids, pis
